# baseline (speedup 1.0000x reference)
import jax
import jax.numpy as jnp
from jax import lax
from jax.experimental import pallas as pl
from jax.experimental.pallas import tpu as pltpu, tpu_sc as plsc

N = 10000
E = 320000
D = 128
NPAD = 10240
NC, NS = 2, 16
NW = NC * NS
CH = 64
NCH = 160
EPAD = NW * NCH * CH
RPS = NPAD // NS


def _sc_deg_body(dstw, zrow, out, didx, ones_v, accd):
    c = lax.axis_index("c")
    s = lax.axis_index("s")
    wid = s * NC + c
    pltpu.sync_copy(zrow, accd.at[pl.ds(s * RPS, RPS)])
    pltpu.sync_copy(dstw.at[wid], didx)
    for i in range(CH // 16):
        ones_v[pl.ds(i * 16, 16)] = jnp.ones((16,), jnp.float32)
    plsc.subcore_barrier()

    def body(j, carry):
        pltpu.sync_copy(ones_v, accd.at[didx.at[j]], add=True)
        return carry

    lax.fori_loop(0, NCH, body, 0)
    plsc.subcore_barrier()
    pltpu.sync_copy(accd.at[pl.ds(s * RPS, RPS)], out.at[c, pl.ds(s * RPS, RPS)])


_deg = pl.kernel(
    _sc_deg_body,
    out_type=jax.ShapeDtypeStruct((NC, NPAD), jnp.float32),
    mesh=plsc.VectorSubcoreMesh(
        core_axis_name="c", subcore_axis_name="s", num_cores=NC, num_subcores=NS
    ),
    scratch_types=[
        pltpu.VMEM((NCH, CH), jnp.int32),
        pltpu.VMEM((CH,), jnp.float32),
        pltpu.VMEM_SHARED((NPAD,), jnp.float32),
    ],
)


RING = 4
GRP = 20
NG = NCH // GRP


def _sc_agg_body(hs, eiw, zrows, out, ib0, ib1, is0, is1, *rest):
    ib = (ib0, ib1)
    isem = (is0, is1)
    gb = rest[0:RING]
    gsem = rest[RING : 2 * RING]
    ssem = rest[2 * RING : 3 * RING]
    acc = rest[3 * RING]
    c = lax.axis_index("c")
    s = lax.axis_index("s")
    wid = s * NC + c
    pltpu.sync_copy(zrows, acc.at[pl.ds(s * RPS, RPS)])
    pltpu.async_copy(eiw.at[wid, pl.ds(0, GRP)], ib[0], isem[0])
    pltpu.make_async_copy(eiw.at[wid, pl.ds(0, GRP)], ib[0], isem[0]).wait()
    pltpu.async_copy(hs.at[ib[0].at[0, 0]], gb[0], gsem[0])
    pltpu.async_copy(hs.at[ib[0].at[1, 0]], gb[1], gsem[1])
    plsc.subcore_barrier()

    def pair(i, carry):
        for gg in range(2):
            g = 2 * i + gg
            for k in range(GRP):
                b = k % RING
                bn = (b + 2) % RING
                pltpu.make_async_copy(hs.at[ib[gg].at[k, 0]], gb[b], gsem[b]).wait()
                pltpu.async_copy(gb[b], acc.at[ib[gg].at[k, 1]], ssem[b], add=True)
                if k >= 2:
                    pltpu.make_async_copy(
                        gb[bn], acc.at[ib[gg].at[k - 2, 1]], ssem[bn]
                    ).wait()
                else:
                    wait_prev = pl.when(g > 0) if gg == 0 else _always

                    @wait_prev
                    def _():
                        pltpu.make_async_copy(
                            gb[bn], acc.at[ib[1 - gg].at[GRP - 2 + k, 1]], ssem[bn]
                        ).wait()

                if k == 2:
                    @pl.when(g + 1 < NG)
                    def _():
                        pltpu.async_copy(
                            eiw.at[wid, pl.ds((g + 1) * GRP, GRP)],
                            ib[1 - gg],
                            isem[1 - gg],
                        )

                if k < GRP - 2:
                    pltpu.async_copy(hs.at[ib[gg].at[k + 2, 0]], gb[bn], gsem[bn])
                else:

                    @pl.when(g < NG - 1)
                    def _():
                        if k == GRP - 2:
                            pltpu.make_async_copy(
                                eiw.at[wid, pl.ds((g + 1) * GRP, GRP)],
                                ib[1 - gg],
                                isem[1 - gg],
                            ).wait()
                        pltpu.async_copy(
                            hs.at[ib[1 - gg].at[k - (GRP - 2), 0]], gb[bn], gsem[bn]
                        )

        return carry

    lax.fori_loop(0, NG // 2, pair, 0)
    for k in (GRP - 2, GRP - 1):
        b = k % RING
        pltpu.make_async_copy(gb[b], acc.at[ib[1].at[k, 1]], ssem[b]).wait()
    plsc.subcore_barrier()
    pltpu.sync_copy(acc.at[pl.ds(s * RPS, RPS)], out.at[c, pl.ds(s * RPS, RPS)])


def _always(f):
    return f()


_agg = pl.kernel(
    _sc_agg_body,
    out_type=jax.ShapeDtypeStruct((NC, NPAD, D), jnp.float32),
    mesh=plsc.VectorSubcoreMesh(
        core_axis_name="c", subcore_axis_name="s", num_cores=NC, num_subcores=NS
    ),
    scratch_types=[
        pltpu.VMEM((GRP, 2, CH), jnp.int32),
        pltpu.VMEM((GRP, 2, CH), jnp.int32),
        pltpu.SemaphoreType.DMA,
        pltpu.SemaphoreType.DMA,
    ]
    + [pltpu.VMEM((CH, D), jnp.float32) for _ in range(RING)]
    + [pltpu.SemaphoreType.DMA for _ in range(2 * RING)]
    + [pltpu.VMEM_SHARED((NPAD, D), jnp.float32)],
)


BLK = 1024


def _tc1_body(x_ref, w_ref, d0_ref, d1_ref, hs_ref, dis_ref):
    deg = d0_ref[...] + d1_ref[...] + 1.0
    dis = lax.rsqrt(deg)
    h = jnp.dot(x_ref[...], w_ref[...], preferred_element_type=jnp.float32)
    hs_ref[...] = h * dis
    dis_ref[...] = dis


def _tc2_body(p0_ref, p1_ref, hs_ref, dis_ref, b_ref, w_ref, out_ref):
    dis = dis_ref[...]
    pre = (p0_ref[...] + p1_ref[...] + hs_ref[...]) * dis + b_ref[...]
    a = jnp.maximum(pre, 0.0)
    out_ref[...] = jnp.dot(a, w_ref[...], preferred_element_type=jnp.float32) * dis


def _tc3_body(q0_ref, q1_ref, hs_ref, dis_ref, b_ref, out_ref):
    pre = (q0_ref[...] + q1_ref[...] + hs_ref[...]) * dis_ref[...] + b_ref[...]
    out_ref[...] = jnp.maximum(pre, 0.0)


_row_spec = pl.BlockSpec((BLK, D), lambda i: (i, 0))
_col_spec = pl.BlockSpec((BLK, 1), lambda i: (i, 0))
_w_spec = pl.BlockSpec((D, D), lambda i: (0, 0))
_b_spec = pl.BlockSpec((1, D), lambda i: (0, 0))

_tc1 = pl.pallas_call(
    _tc1_body,
    grid=(NPAD // BLK,),
    in_specs=[_row_spec, _w_spec, _col_spec, _col_spec],
    out_specs=[_row_spec, _col_spec],
    out_shape=[
        jax.ShapeDtypeStruct((NPAD, D), jnp.float32),
        jax.ShapeDtypeStruct((NPAD, 1), jnp.float32),
    ],
)

_tc2 = pl.pallas_call(
    _tc2_body,
    grid=(NPAD // BLK,),
    in_specs=[_row_spec, _row_spec, _row_spec, _col_spec, _b_spec, _w_spec],
    out_specs=_row_spec,
    out_shape=jax.ShapeDtypeStruct((NPAD, D), jnp.float32),
)

_tc3 = pl.pallas_call(
    _tc3_body,
    grid=(NPAD // BLK,),
    in_specs=[_row_spec, _row_spec, _row_spec, _col_spec, _b_spec],
    out_specs=_row_spec,
    out_shape=jax.ShapeDtypeStruct((NPAD, D), jnp.float32),
)


def kernel(x, edge_index, batch, W1, b1, W2, b2):
    src = edge_index[0].astype(jnp.int32)
    dst = edge_index[1].astype(jnp.int32)
    pad = EPAD - E
    srcw = jnp.concatenate([src, jnp.zeros((pad,), jnp.int32)]).reshape(NW, NCH, CH)
    dstw = jnp.concatenate([dst, jnp.full((pad,), N, jnp.int32)]).reshape(NW, NCH, CH)
    eiw = jnp.stack([srcw, dstw], axis=2)
    xp = jnp.pad(x, ((0, NPAD - N), (0, 0)))
    zrows = jnp.zeros((RPS, D), jnp.float32)
    zrow = jnp.zeros((RPS,), jnp.float32)

    degp = _deg(dstw, zrow)
    d0 = degp[0].reshape(NPAD, 1)
    d1 = degp[1].reshape(NPAD, 1)
    hs1, dis = _tc1(xp, W1, d0, d1)
    p = _agg(hs1, eiw, zrows)
    hs2 = _tc2(p[0], p[1], hs1, dis, b1.reshape(1, D), W2)
    q = _agg(hs2, eiw, zrows)
    out = _tc3(q[0], q[1], hs2, dis, b2.reshape(1, D))
    return out[:N]


# speedup vs baseline: 3.1631x; 3.1631x over previous
import jax
import jax.numpy as jnp
from jax import lax
from jax.experimental import pallas as pl
from jax.experimental.pallas import tpu as pltpu, tpu_sc as plsc

N = 10000
E = 320000
D = 128
NPAD = 10240
NC, NS = 2, 16
NW = NC * NS
CH = 128
NCH = 80
EPAD = NW * NCH * CH
RPS = NPAD // NS


def _sc_deg_body(dstw, zrow, out, didx, ones_v, accd):
    c = lax.axis_index("c")
    s = lax.axis_index("s")
    wid = s * NC + c
    pltpu.sync_copy(zrow, accd.at[pl.ds(s * RPS, RPS)])
    pltpu.sync_copy(dstw.at[wid], didx)
    for i in range(CH // 16):
        ones_v[pl.ds(i * 16, 16)] = jnp.ones((16,), jnp.float32)
    plsc.subcore_barrier()

    def body(j, carry):
        pltpu.sync_copy(ones_v, accd.at[didx.at[j]], add=True)
        return carry

    lax.fori_loop(0, NCH, body, 0)
    plsc.subcore_barrier()
    pltpu.sync_copy(accd.at[pl.ds(s * RPS, RPS)], out.at[c, pl.ds(s * RPS, RPS)])


_deg = pl.kernel(
    _sc_deg_body,
    out_type=jax.ShapeDtypeStruct((NC, NPAD), jnp.float32),
    mesh=plsc.VectorSubcoreMesh(
        core_axis_name="c", subcore_axis_name="s", num_cores=NC, num_subcores=NS
    ),
    scratch_types=[
        pltpu.VMEM((NCH, CH), jnp.int32),
        pltpu.VMEM((CH,), jnp.float32),
        pltpu.VMEM_SHARED((NPAD,), jnp.float32),
    ],
)


GRP = 10
NG = NCH // GRP


def _sc_agg_body(hs, eiw, zrows, out, ib0, ib1, gb0, gb1, is0, is1, gs0, gs1, acc):
    ib = (ib0, ib1)
    gb = (gb0, gb1)
    isem = (is0, is1)
    gsem = (gs0, gs1)
    c = lax.axis_index("c")
    s = lax.axis_index("s")
    wid = s * NC + c
    pltpu.sync_copy(zrows, acc.at[pl.ds(s * RPS, RPS)])
    pltpu.async_copy(eiw.at[wid, pl.ds(0, GRP)], ib[0], isem[0])
    pltpu.async_copy(eiw.at[wid, pl.ds(GRP, GRP)], ib[1], isem[1])
    pltpu.make_async_copy(eiw.at[wid, pl.ds(0, GRP)], ib[0], isem[0]).wait()
    pltpu.async_copy(hs.at[ib[0].at[0, 0]], gb[0], gsem[0])
    plsc.subcore_barrier()

    def pair(i, carry):
        for gg in range(2):
            g = 2 * i + gg
            for k in range(GRP):
                b = k % 2
                if k < GRP - 1:
                    pltpu.async_copy(hs.at[ib[gg].at[k + 1, 0]], gb[1 - b], gsem[1 - b])
                else:

                    @pl.when(g < NG - 1)
                    def _():
                        pltpu.make_async_copy(
                            eiw.at[wid, pl.ds((g + 1) * GRP, GRP)],
                            ib[1 - gg],
                            isem[1 - gg],
                        ).wait()
                        pltpu.async_copy(
                            hs.at[ib[1 - gg].at[0, 0]], gb[1 - b], gsem[1 - b]
                        )

                pltpu.make_async_copy(hs.at[ib[gg].at[k, 0]], gb[b], gsem[b]).wait()
                pltpu.sync_copy(gb[b], acc.at[ib[gg].at[k, 1]], add=True)

            @pl.when(g + 2 < NG)
            def _():
                pltpu.async_copy(
                    eiw.at[wid, pl.ds((g + 2) * GRP, GRP)], ib[gg], isem[gg]
                )

        return carry

    lax.fori_loop(0, NG // 2, pair, 0)
    plsc.subcore_barrier()
    pltpu.sync_copy(acc.at[pl.ds(s * RPS, RPS)], out.at[c, pl.ds(s * RPS, RPS)])


_agg = pl.kernel(
    _sc_agg_body,
    out_type=jax.ShapeDtypeStruct((NC, NPAD, D), jnp.float32),
    mesh=plsc.VectorSubcoreMesh(
        core_axis_name="c", subcore_axis_name="s", num_cores=NC, num_subcores=NS
    ),
    scratch_types=[
        pltpu.VMEM((GRP, 2, CH), jnp.int32),
        pltpu.VMEM((GRP, 2, CH), jnp.int32),
        pltpu.VMEM((CH, D), jnp.float32),
        pltpu.VMEM((CH, D), jnp.float32),
        pltpu.SemaphoreType.DMA,
        pltpu.SemaphoreType.DMA,
        pltpu.SemaphoreType.DMA,
        pltpu.SemaphoreType.DMA,
        pltpu.VMEM_SHARED((NPAD, D), jnp.float32),
    ],
)


BLK = 1024


def _tc1_body(x_ref, w_ref, d0_ref, d1_ref, hs_ref, dis_ref):
    deg = d0_ref[...] + d1_ref[...] + 1.0
    dis = lax.rsqrt(deg)
    h = jnp.dot(x_ref[...], w_ref[...], preferred_element_type=jnp.float32)
    hs_ref[...] = h * dis
    dis_ref[...] = dis


def _tc2_body(p0_ref, p1_ref, hs_ref, dis_ref, b_ref, w_ref, out_ref):
    dis = dis_ref[...]
    pre = (p0_ref[...] + p1_ref[...] + hs_ref[...]) * dis + b_ref[...]
    a = jnp.maximum(pre, 0.0)
    out_ref[...] = jnp.dot(a, w_ref[...], preferred_element_type=jnp.float32) * dis


def _tc3_body(q0_ref, q1_ref, hs_ref, dis_ref, b_ref, out_ref):
    pre = (q0_ref[...] + q1_ref[...] + hs_ref[...]) * dis_ref[...] + b_ref[...]
    out_ref[...] = jnp.maximum(pre, 0.0)


_row_spec = pl.BlockSpec((BLK, D), lambda i: (i, 0))
_col_spec = pl.BlockSpec((BLK, 1), lambda i: (i, 0))
_w_spec = pl.BlockSpec((D, D), lambda i: (0, 0))
_b_spec = pl.BlockSpec((1, D), lambda i: (0, 0))

_tc1 = pl.pallas_call(
    _tc1_body,
    grid=(NPAD // BLK,),
    in_specs=[_row_spec, _w_spec, _col_spec, _col_spec],
    out_specs=[_row_spec, _col_spec],
    out_shape=[
        jax.ShapeDtypeStruct((NPAD, D), jnp.float32),
        jax.ShapeDtypeStruct((NPAD, 1), jnp.float32),
    ],
)

_tc2 = pl.pallas_call(
    _tc2_body,
    grid=(NPAD // BLK,),
    in_specs=[_row_spec, _row_spec, _row_spec, _col_spec, _b_spec, _w_spec],
    out_specs=_row_spec,
    out_shape=jax.ShapeDtypeStruct((NPAD, D), jnp.float32),
)

_tc3 = pl.pallas_call(
    _tc3_body,
    grid=(NPAD // BLK,),
    in_specs=[_row_spec, _row_spec, _row_spec, _col_spec, _b_spec],
    out_specs=_row_spec,
    out_shape=jax.ShapeDtypeStruct((NPAD, D), jnp.float32),
)


def kernel(x, edge_index, batch, W1, b1, W2, b2):
    src = edge_index[0].astype(jnp.int32)
    dst = edge_index[1].astype(jnp.int32)
    pad = EPAD - E
    pad_iota = jnp.arange(pad, dtype=jnp.int32)
    pad_src = pad_iota % N
    pad_dst = N + pad_iota % (NPAD - N)
    srcw = jnp.concatenate([src, pad_src]).reshape(NW, NCH, CH)
    dstw = jnp.concatenate([dst, pad_dst]).reshape(NW, NCH, CH)
    eiw = jnp.stack([srcw, dstw], axis=2)
    xp = jnp.pad(x, ((0, NPAD - N), (0, 0)))
    zrows = jnp.zeros((RPS, D), jnp.float32)
    zrow = jnp.zeros((RPS,), jnp.float32)

    degp = _deg(dstw, zrow)
    d0 = degp[0].reshape(NPAD, 1)
    d1 = degp[1].reshape(NPAD, 1)
    hs1, dis = _tc1(xp, W1, d0, d1)
    p = _agg(hs1, eiw, zrows)
    hs2 = _tc2(p[0], p[1], hs1, dis, b1.reshape(1, D), W2)
    q = _agg(hs2, eiw, zrows)
    out = _tc3(q[0], q[1], hs2, dis, b2.reshape(1, D))
    return out[:N]


# speedup vs baseline: 3.2082x; 1.0143x over previous
import jax
import jax.numpy as jnp
from jax import lax
from jax.experimental import pallas as pl
from jax.experimental.pallas import tpu as pltpu, tpu_sc as plsc

N = 10000
E = 320000
D = 128
NPAD = 10240
NC, NS = 2, 16
NW = NC * NS
CH = 128
NCH = 80
EPAD = NW * NCH * CH
RPS = NPAD // NS


def _sc_deg_body(dstw, zrow, out, didx, ones_v, accd):
    c = lax.axis_index("c")
    s = lax.axis_index("s")
    wid = s * NC + c
    pltpu.sync_copy(zrow, accd.at[pl.ds(s * RPS, RPS)])
    pltpu.sync_copy(dstw.at[wid], didx)
    for i in range(CH // 16):
        ones_v[pl.ds(i * 16, 16)] = jnp.ones((16,), jnp.float32)
    plsc.subcore_barrier()

    def body(j, carry):
        pltpu.sync_copy(ones_v, accd.at[didx.at[j]], add=True)
        return carry

    lax.fori_loop(0, NCH, body, 0)
    plsc.subcore_barrier()
    pltpu.sync_copy(accd.at[pl.ds(s * RPS, RPS)], out.at[c, pl.ds(s * RPS, RPS)])


_deg = pl.kernel(
    _sc_deg_body,
    out_type=jax.ShapeDtypeStruct((NC, NPAD), jnp.float32),
    mesh=plsc.VectorSubcoreMesh(
        core_axis_name="c", subcore_axis_name="s", num_cores=NC, num_subcores=NS
    ),
    scratch_types=[
        pltpu.VMEM((NCH, CH), jnp.int32),
        pltpu.VMEM((CH,), jnp.float32),
        pltpu.VMEM_SHARED((NPAD,), jnp.float32),
    ],
)


GRP = 10
NG = NCH // GRP


def _sc_agg_body(hs, eiw, zrows, out, ib0, ib1, gb0, gb1, is0, is1, gs0, gs1, acc):
    ib = (ib0, ib1)
    gb = (gb0, gb1)
    isem = (is0, is1)
    gsem = (gs0, gs1)
    c = lax.axis_index("c")
    s = lax.axis_index("s")
    wid = s * NC + c
    pltpu.sync_copy(zrows, acc.at[pl.ds(s * RPS, RPS)])
    pltpu.async_copy(eiw.at[wid, pl.ds(0, GRP)], ib[0], isem[0])
    pltpu.async_copy(eiw.at[wid, pl.ds(GRP, GRP)], ib[1], isem[1])
    pltpu.make_async_copy(eiw.at[wid, pl.ds(0, GRP)], ib[0], isem[0]).wait()
    pltpu.async_copy(hs.at[ib[0].at[0, 0]], gb[0], gsem[0])
    plsc.subcore_barrier()

    def pair(i, carry):
        for gg in range(2):
            g = 2 * i + gg
            for k in range(GRP):
                b = k % 2
                if k < GRP - 1:
                    pltpu.async_copy(hs.at[ib[gg].at[k + 1, 0]], gb[1 - b], gsem[1 - b])
                else:

                    @pl.when(g < NG - 1)
                    def _():
                        pltpu.make_async_copy(
                            eiw.at[wid, pl.ds((g + 1) * GRP, GRP)],
                            ib[1 - gg],
                            isem[1 - gg],
                        ).wait()
                        pltpu.async_copy(
                            hs.at[ib[1 - gg].at[0, 0]], gb[1 - b], gsem[1 - b]
                        )

                pltpu.make_async_copy(hs.at[ib[gg].at[k, 0]], gb[b], gsem[b]).wait()
                pltpu.sync_copy(gb[b], acc.at[ib[gg].at[k, 1]], add=True)

            @pl.when(g + 2 < NG)
            def _():
                pltpu.async_copy(
                    eiw.at[wid, pl.ds((g + 2) * GRP, GRP)], ib[gg], isem[gg]
                )

        return carry

    lax.fori_loop(0, NG // 2, pair, 0)
    plsc.subcore_barrier()
    pltpu.sync_copy(acc.at[pl.ds(s * RPS, RPS)], out.at[c, pl.ds(s * RPS, RPS)])


_agg = pl.kernel(
    _sc_agg_body,
    out_type=jax.ShapeDtypeStruct((NC, NPAD, D), jnp.float32),
    mesh=plsc.VectorSubcoreMesh(
        core_axis_name="c", subcore_axis_name="s", num_cores=NC, num_subcores=NS
    ),
    scratch_types=[
        pltpu.VMEM((GRP, 2, CH), jnp.int32),
        pltpu.VMEM((GRP, 2, CH), jnp.int32),
        pltpu.VMEM((CH, D), jnp.float32),
        pltpu.VMEM((CH, D), jnp.float32),
        pltpu.SemaphoreType.DMA,
        pltpu.SemaphoreType.DMA,
        pltpu.SemaphoreType.DMA,
        pltpu.SemaphoreType.DMA,
        pltpu.VMEM_SHARED((NPAD, D), jnp.float32),
    ],
)


BLK = 1024


def _mm1_body(x_ref, w_ref, h_ref):
    h_ref[...] = jnp.dot(x_ref[...], w_ref[...], preferred_element_type=jnp.float32)


def _tc1_body(h_ref, d0_ref, d1_ref, hs_ref, dis_ref):
    deg = d0_ref[...] + d1_ref[...] + 1.0
    dis = lax.rsqrt(deg)
    hs_ref[...] = h_ref[...] * dis
    dis_ref[...] = dis


def _tc2_body(p0_ref, p1_ref, hs_ref, dis_ref, b_ref, w_ref, out_ref):
    dis = dis_ref[...]
    pre = (p0_ref[...] + p1_ref[...] + hs_ref[...]) * dis + b_ref[...]
    a = jnp.maximum(pre, 0.0)
    out_ref[...] = jnp.dot(a, w_ref[...], preferred_element_type=jnp.float32) * dis


def _tc3_body(q0_ref, q1_ref, hs_ref, dis_ref, b_ref, out_ref):
    pre = (q0_ref[...] + q1_ref[...] + hs_ref[...]) * dis_ref[...] + b_ref[...]
    out_ref[...] = jnp.maximum(pre, 0.0)


_row_spec = pl.BlockSpec((BLK, D), lambda i: (i, 0))
_col_spec = pl.BlockSpec((BLK, 1), lambda i: (i, 0))
_w_spec = pl.BlockSpec((D, D), lambda i: (0, 0))
_b_spec = pl.BlockSpec((1, D), lambda i: (0, 0))

_mm1 = pl.pallas_call(
    _mm1_body,
    grid=(NPAD // BLK,),
    in_specs=[_row_spec, _w_spec],
    out_specs=_row_spec,
    out_shape=jax.ShapeDtypeStruct((NPAD, D), jnp.float32),
)

_tc1 = pl.pallas_call(
    _tc1_body,
    grid=(NPAD // BLK,),
    in_specs=[_row_spec, _col_spec, _col_spec],
    out_specs=[_row_spec, _col_spec],
    out_shape=[
        jax.ShapeDtypeStruct((NPAD, D), jnp.float32),
        jax.ShapeDtypeStruct((NPAD, 1), jnp.float32),
    ],
)

_tc2 = pl.pallas_call(
    _tc2_body,
    grid=(NPAD // BLK,),
    in_specs=[_row_spec, _row_spec, _row_spec, _col_spec, _b_spec, _w_spec],
    out_specs=_row_spec,
    out_shape=jax.ShapeDtypeStruct((NPAD, D), jnp.float32),
)

_OBLK = 1000
_orow_spec = pl.BlockSpec((_OBLK, D), lambda i: (i, 0))
_ocol_spec = pl.BlockSpec((_OBLK, 1), lambda i: (i, 0))
_ob_spec = pl.BlockSpec((1, D), lambda i: (0, 0))

_tc3 = pl.pallas_call(
    _tc3_body,
    grid=(N // _OBLK,),
    in_specs=[_orow_spec, _orow_spec, _orow_spec, _ocol_spec, _ob_spec],
    out_specs=_orow_spec,
    out_shape=jax.ShapeDtypeStruct((N, D), jnp.float32),
)


def kernel(x, edge_index, batch, W1, b1, W2, b2):
    src = edge_index[0].astype(jnp.int32)
    dst = edge_index[1].astype(jnp.int32)
    pad = EPAD - E
    pad_iota = jnp.arange(pad, dtype=jnp.int32)
    pad_src = pad_iota % N
    pad_dst = N + pad_iota % (NPAD - N)
    srcw = jnp.concatenate([src, pad_src]).reshape(NW, NCH, CH)
    dstw = jnp.concatenate([dst, pad_dst]).reshape(NW, NCH, CH)
    eiw = jnp.stack([srcw, dstw], axis=2)
    xp = jnp.pad(x, ((0, NPAD - N), (0, 0)))
    zrows = jnp.zeros((RPS, D), jnp.float32)
    zrow = jnp.zeros((RPS,), jnp.float32)

    h1 = _mm1(xp, W1)
    degp = _deg(dstw, zrow)
    d0 = degp[0].reshape(NPAD, 1)
    d1 = degp[1].reshape(NPAD, 1)
    hs1, dis = _tc1(h1, d0, d1)
    p = _agg(hs1, eiw, zrows)
    hs2 = _tc2(p[0], p[1], hs1, dis, b1.reshape(1, D), W2)
    q = _agg(hs2, eiw, zrows)
    return _tc3(q[0], q[1], hs2, dis, b2.reshape(1, D))
